# baseline (speedup 1.0000x reference)
import jax
import jax.numpy as jnp
from jax import lax
from jax.experimental import pallas as pl
from jax.experimental.pallas import tpu as pltpu
from jax.experimental.pallas import tpu_sc as plsc

N = 50000
E = 800000
G = 512
NP = 50176
EP = 819200
EPR = EP // 8
EPW = EP // 32
K = 4096
KD = 5120
GP = 520
GW = 8

_f32 = jnp.float32
_i32 = jnp.int32

_SC_PARAMS = pltpu.CompilerParams(use_tc_tiling_on_sc=False,
                                  needs_layout_passes=False)


def _mesh():
    return plsc.VectorSubcoreMesh(core_axis_name="c", subcore_axis_name="s")


def _kw():
    return dict(mesh=_mesh(), compiler_params=_SC_PARAMS)


def _wid():
    return lax.axis_index("c") * 16 + lax.axis_index("s")


def _zero(buf, nwords):
    z = jnp.zeros((16,), _f32)

    @plsc.parallel_loop(0, nwords // 16, unroll=8)
    def _(i):
        buf[pl.ds(i * 16, 16)] = z


def _deg_body(dst_hbm, out_hbm, acc, b0, b1, s0, s1):
    wid = _wid()
    _zero(acc, NP)
    ones16 = jnp.full((16,), 1.0, _f32)
    base = wid * EPW
    nch = EPW // KD
    bufs = (b0, b1)
    sems = (s0, s1)

    def fire(ch):
        pltpu.async_copy(dst_hbm.at[pl.ds(base + ch * KD, KD)],
                         bufs[ch % 2], sems[ch % 2])

    fire(0)
    for ch in range(nch):
        buf, sem = bufs[ch % 2], sems[ch % 2]
        pltpu.make_async_copy(dst_hbm.at[pl.ds(base + ch * KD, KD)],
                              buf, sem).wait()
        if ch + 1 < nch:
            fire(ch + 1)

        @plsc.parallel_loop(0, KD // 16, unroll=16)
        def _(i):
            plsc.addupdate_scatter(acc, [buf[pl.ds(i * 16, 16)]], ones16)
    pltpu.sync_copy(acc, out_hbm.at[wid])


def _deg_call(dstp):
    return pl.kernel(
        _deg_body,
        out_type=jax.ShapeDtypeStruct((32, NP), _f32),
        scratch_types=[
            pltpu.VMEM((NP,), _f32),
            pltpu.VMEM((KD,), _i32),
            pltpu.VMEM((KD,), _i32),
            pltpu.SemaphoreType.DMA,
            pltpu.SemaphoreType.DMA,
        ],
        **_kw(),
    )(dstp)


def _step_body(ucols_hbm, src_hbm, dst_hbm, out_hbm,
               ucol, acc, sb0, db0, sb1, db1, ss0, sd0, ss1, sd1):
    wid = _wid()
    col = lax.rem(wid, 4)
    rng = wid // 4
    pltpu.sync_copy(ucols_hbm.at[col], ucol)
    _zero(acc, NP)
    base = rng * EPR
    nch = EPR // K
    sbufs = (sb0, sb1)
    dbufs = (db0, db1)
    ssems = (ss0, ss1)
    dsems = (sd0, sd1)

    def fire(ch):
        b = ch % 2
        pltpu.async_copy(src_hbm.at[pl.ds(base + ch * K, K)],
                         sbufs[b], ssems[b])
        pltpu.async_copy(dst_hbm.at[pl.ds(base + ch * K, K)],
                         dbufs[b], dsems[b])

    fire(0)
    for ch in range(nch):
        b = ch % 2
        pltpu.make_async_copy(src_hbm.at[pl.ds(base + ch * K, K)],
                              sbufs[b], ssems[b]).wait()
        pltpu.make_async_copy(dst_hbm.at[pl.ds(base + ch * K, K)],
                              dbufs[b], dsems[b]).wait()
        if ch + 1 < nch:
            fire(ch + 1)
        sbuf, dbuf = sbufs[b], dbufs[b]

        @plsc.parallel_loop(0, K // 16, unroll=32)
        def _(i):
            sl = pl.ds(i * 16, 16)
            g = plsc.load_gather(ucol, [sbuf[sl]])
            plsc.addupdate_scatter(acc, [dbuf[sl]], g)
    pltpu.sync_copy(acc, out_hbm.at[rng, col])


def _step_call(ucols, srcp, dstp):
    return pl.kernel(
        _step_body,
        out_type=jax.ShapeDtypeStruct((8, 4, NP), _f32),
        scratch_types=[
            pltpu.VMEM((NP,), _f32),
            pltpu.VMEM((NP,), _f32),
            pltpu.VMEM((K,), _i32),
            pltpu.VMEM((K,), _i32),
            pltpu.VMEM((K,), _i32),
            pltpu.VMEM((K,), _i32),
            pltpu.SemaphoreType.DMA,
            pltpu.SemaphoreType.DMA,
            pltpu.SemaphoreType.DMA,
            pltpu.SemaphoreType.DMA,
        ],
        **_kw(),
    )(ucols, srcp, dstp)


_MR = NP // 8


def _merge_body(parts_hbm, ucols_hbm, sm_hbm, d1_hbm, un_hbm, zc_hbm,
                pbuf, ubuf, sbuf, dbuf, sumb, obuf, zbuf):
    wid = _wid()
    col = lax.rem(wid, 4)
    nrng = wid // 4
    off = nrng * _MR
    for r in range(8):
        pltpu.sync_copy(parts_hbm.at[r, col, pl.ds(off, _MR)], pbuf.at[r])
    pltpu.sync_copy(ucols_hbm.at[col, pl.ds(off, _MR)], ubuf)
    pltpu.sync_copy(sm_hbm.at[pl.ds(off, _MR)], sbuf)

    @plsc.parallel_loop(0, _MR // 16, unroll=4)
    def _(i):
        sl = pl.ds(i * 16, 16)
        sm = ubuf[sl]
        for r in range(8):
            sm = sm + pbuf[r, sl]
        sumb[sl] = sm
        obuf[sl] = sbuf[sl] * sm
    pltpu.sync_copy(obuf, un_hbm.at[col, pl.ds(off, _MR)])

    @pl.when(col == 3)
    def _():
        pltpu.sync_copy(d1_hbm.at[pl.ds(off, _MR)], dbuf)

        @plsc.parallel_loop(0, _MR // 16, unroll=8)
        def _(i):
            sl = pl.ds(i * 16, 16)
            zbuf[sl] = dbuf[sl] * sumb[sl]
        pltpu.sync_copy(zbuf, zc_hbm.at[pl.ds(off, _MR)])


def _merge_call(parts, ucols, smain, d1):
    return pl.kernel(
        _merge_body,
        out_type=(jax.ShapeDtypeStruct((4, NP), _f32),
                  jax.ShapeDtypeStruct((NP,), _f32)),
        scratch_types=[
            pltpu.VMEM((8, _MR), _f32),
            pltpu.VMEM((_MR,), _f32),
            pltpu.VMEM((_MR,), _f32),
            pltpu.VMEM((_MR,), _f32),
            pltpu.VMEM((_MR,), _f32),
            pltpu.VMEM((_MR,), _f32),
            pltpu.VMEM((_MR,), _f32),
        ],
        **_kw(),
    )(parts, ucols, smain, d1)


_SR = NP // 32


def _seg_body(z3_hbm, zc1_hbm, zc2_hbm, bp_hbm, out_hbm,
              accf, bbuf, v0, v1, v2, v3, c1b, c2b):
    wid = _wid()
    _zero(accf, GP * GW)
    off = wid * _SR
    pltpu.sync_copy(bp_hbm.at[pl.ds(off, _SR)], bbuf)
    for k, vb in enumerate((v0, v1, v2, v3)):
        pltpu.sync_copy(z3_hbm.at[k, pl.ds(off, _SR)], vb)
    pltpu.sync_copy(zc1_hbm.at[pl.ds(off, _SR)], c1b)
    pltpu.sync_copy(zc2_hbm.at[pl.ds(off, _SR)], c2b)
    ones16 = jnp.full((16,), 1.0, _f32)

    @plsc.parallel_loop(0, _SR // 16, unroll=2)
    def _(i):
        sl = pl.ds(i * 16, 16)
        ix = bbuf[sl] * GW
        for cst, vb in ((0, v0), (1, v1), (2, v2), (3, v3),
                        (4, c1b), (5, c2b)):
            plsc.addupdate_scatter(accf, [ix + cst], vb[sl])
        plsc.addupdate_scatter(accf, [ix + 6], ones16)
    pltpu.sync_copy(accf, out_hbm.at[wid])


def _seg_call(z3c, zc1, zc2, bp):
    return pl.kernel(
        _seg_body,
        out_type=jax.ShapeDtypeStruct((32, GP * GW), _f32),
        scratch_types=[
            pltpu.VMEM((GP * GW,), _f32),
            pltpu.VMEM((_SR,), _i32),
            pltpu.VMEM((_SR,), _f32),
            pltpu.VMEM((_SR,), _f32),
            pltpu.VMEM((_SR,), _f32),
            pltpu.VMEM((_SR,), _f32),
            pltpu.VMEM((_SR,), _f32),
            pltpu.VMEM((_SR,), _f32),
        ],
        **_kw(),
    )(z3c, zc1, zc2, bp)


_RB = NP // 8


def _prep_body(degp_ref, xt_ref, u1_ref, d1_ref, d2_ref):
    deg = jnp.sum(degp_ref[...], axis=0, keepdims=True) + 1.0
    dinv = lax.rsqrt(deg)
    dinv = dinv * (1.5 - 0.5 * deg * dinv * dinv)
    ones = jnp.ones_like(deg)
    u1_ref[...] = jnp.concatenate([xt_ref[...], ones], axis=0) * dinv
    d1_ref[...] = dinv
    d2_ref[...] = dinv * dinv


def _prep_call(degp, xt):
    return pl.pallas_call(
        _prep_body,
        grid=(NP // _RB,),
        in_specs=[
            pl.BlockSpec((32, _RB), lambda i: (0, i)),
            pl.BlockSpec((3, _RB), lambda i: (0, i)),
        ],
        out_specs=[
            pl.BlockSpec((4, _RB), lambda i: (0, i)),
            pl.BlockSpec((1, _RB), lambda i: (0, i)),
            pl.BlockSpec((1, _RB), lambda i: (0, i)),
        ],
        out_shape=[
            jax.ShapeDtypeStruct((4, NP), _f32),
            jax.ShapeDtypeStruct((1, NP), _f32),
            jax.ShapeDtypeStruct((1, NP), _f32),
        ],
    )(degp, xt)


def _final_body(segp_ref, w1_ref, b1_ref, w2_ref, b2_ref,
                w3_ref, b3_ref, out_ref):
    s = jnp.sum(segp_ref[...], axis=0)
    m3 = s[:512, 0:3]
    z1s = s[:512, 4:5]
    z2s = s[:512, 5:6]
    cnt = jnp.maximum(s[:512, 6:7], 1.0)
    w12 = jnp.dot(w1_ref[...], w2_ref[...], preferred_element_type=_f32, precision=lax.Precision.HIGHEST)
    w123 = jnp.dot(w12, w3_ref[...], preferred_element_type=_f32, precision=lax.Precision.HIGHEST)
    v1 = jnp.dot(jnp.dot(b1_ref[...], w2_ref[...],
                         preferred_element_type=_f32, precision=lax.Precision.HIGHEST),
                 w3_ref[...], preferred_element_type=_f32, precision=lax.Precision.HIGHEST)
    v2 = jnp.dot(b2_ref[...], w3_ref[...], preferred_element_type=_f32, precision=lax.Precision.HIGHEST)
    num = jnp.dot(m3, w123, preferred_element_type=_f32, precision=lax.Precision.HIGHEST)
    num = num + z2s * v1 + z1s * v2
    out_ref[...] = num / cnt + b3_ref[...]


def _final_call(segp, W1, b1, W2, b2, W3, b3):
    return pl.pallas_call(
        _final_body,
        out_shape=jax.ShapeDtypeStruct((G, 192), _f32),
    )(segp, W1, b1, W2, b2, W3, b3)


def kernel(node_features, edge_indices, batch_indices,
           W1, b1, W2, b2, W3, b3):
    epad = N + jnp.arange(EP - E, dtype=_i32) % (NP - N)
    srcp = jnp.concatenate([edge_indices[0], epad])
    dstp = jnp.concatenate([edge_indices[1], epad])
    xt = jnp.pad(node_features, ((0, NP - N), (0, 0))).T
    bp = jnp.pad(batch_indices, (0, NP - N), constant_values=G)

    degp = _deg_call(dstp)
    u1c, d1r, d2r = _prep_call(degp, xt)
    d1 = d1r.reshape(NP)
    d2 = d2r.reshape(NP)

    uc = u1c
    zcs = []
    for t in range(3):
        parts = _step_call(uc, srcp, dstp)
        smain = d2 if t < 2 else d1
        uc, zc = _merge_call(parts, uc, smain, d1)
        zcs.append(zc)

    segp = _seg_call(uc, zcs[0], zcs[1], bp).reshape(32, GP, GW)
    out = _final_call(segp, W1, b1.reshape(1, -1), W2,
                      b2.reshape(1, -1), W3, b3.reshape(1, -1))
    return out


# speedup vs baseline: 1.1055x; 1.1055x over previous
import jax
import jax.numpy as jnp
from jax import lax
from jax.experimental import pallas as pl
from jax.experimental.pallas import tpu as pltpu
from jax.experimental.pallas import tpu_sc as plsc

N = 50000
E = 800000
G = 512
NP = 50176
EP = 819200
EPR = EP // 8
EPW = EP // 32
K = 4096
KD = 5120
GP = 520
GW = 8

_f32 = jnp.float32
_i32 = jnp.int32

_SC_PARAMS = pltpu.CompilerParams(use_tc_tiling_on_sc=False,
                                  needs_layout_passes=False)


def _mesh():
    return plsc.VectorSubcoreMesh(core_axis_name="c", subcore_axis_name="s")


def _kw():
    return dict(mesh=_mesh(), compiler_params=_SC_PARAMS)


def _wid():
    return lax.axis_index("c") * 16 + lax.axis_index("s")


def _zero(buf, nwords):
    z = jnp.zeros((16,), _f32)

    @plsc.parallel_loop(0, nwords // 16, unroll=8)
    def _(i):
        buf[pl.ds(i * 16, 16)] = z


def _deg_body(dst_hbm, out_hbm, acc, b0, b1, s0, s1):
    wid = _wid()
    _zero(acc, NP)
    ones16 = jnp.full((16,), 1.0, _f32)
    base = wid * EPW
    nch = EPW // KD
    bufs = (b0, b1)
    sems = (s0, s1)

    def fire(ch):
        pltpu.async_copy(dst_hbm.at[pl.ds(base + ch * KD, KD)],
                         bufs[ch % 2], sems[ch % 2])

    fire(0)
    for ch in range(nch):
        buf, sem = bufs[ch % 2], sems[ch % 2]
        pltpu.make_async_copy(dst_hbm.at[pl.ds(base + ch * KD, KD)],
                              buf, sem).wait()
        if ch + 1 < nch:
            fire(ch + 1)

        @plsc.parallel_loop(0, KD // 16, unroll=16)
        def _(i):
            plsc.addupdate_scatter(acc, [buf[pl.ds(i * 16, 16)]], ones16)
    pltpu.sync_copy(acc, out_hbm.at[wid])


def _deg_call(dstp):
    return pl.kernel(
        _deg_body,
        out_type=jax.ShapeDtypeStruct((32, NP), _f32),
        scratch_types=[
            pltpu.VMEM((NP,), _f32),
            pltpu.VMEM((KD,), _i32),
            pltpu.VMEM((KD,), _i32),
            pltpu.SemaphoreType.DMA,
            pltpu.SemaphoreType.DMA,
        ],
        **_kw(),
    )(dstp)


def _step_body(ucols_hbm, src_hbm, dst_hbm, out_hbm,
               ucol, acc, sb0, db0, sb1, db1, ss0, sd0, ss1, sd1):
    wid = _wid()
    col = lax.rem(wid, 4)
    rng = wid // 4
    pltpu.sync_copy(ucols_hbm.at[col], ucol)
    _zero(acc, NP)
    base = rng * EPR
    nch = EPR // K
    sbufs = (sb0, sb1)
    dbufs = (db0, db1)
    ssems = (ss0, ss1)
    dsems = (sd0, sd1)

    def fire(ch):
        b = ch % 2
        pltpu.async_copy(src_hbm.at[pl.ds(base + ch * K, K)],
                         sbufs[b], ssems[b])
        pltpu.async_copy(dst_hbm.at[pl.ds(base + ch * K, K)],
                         dbufs[b], dsems[b])

    fire(0)
    for ch in range(nch):
        b = ch % 2
        pltpu.make_async_copy(src_hbm.at[pl.ds(base + ch * K, K)],
                              sbufs[b], ssems[b]).wait()
        pltpu.make_async_copy(dst_hbm.at[pl.ds(base + ch * K, K)],
                              dbufs[b], dsems[b]).wait()
        if ch + 1 < nch:
            fire(ch + 1)
        sbuf, dbuf = sbufs[b], dbufs[b]

        @plsc.parallel_loop(0, K // 16, unroll=16)
        def _(i):
            sl = pl.ds(i * 16, 16)
            g = plsc.load_gather(ucol, [sbuf[sl]])
            plsc.addupdate_scatter(acc, [dbuf[sl]], g)
    pltpu.sync_copy(acc, out_hbm.at[rng, col])


def _step_call(ucols, srcp, dstp):
    return pl.kernel(
        _step_body,
        out_type=jax.ShapeDtypeStruct((8, 4, NP), _f32),
        scratch_types=[
            pltpu.VMEM((NP,), _f32),
            pltpu.VMEM((NP,), _f32),
            pltpu.VMEM((K,), _i32),
            pltpu.VMEM((K,), _i32),
            pltpu.VMEM((K,), _i32),
            pltpu.VMEM((K,), _i32),
            pltpu.SemaphoreType.DMA,
            pltpu.SemaphoreType.DMA,
            pltpu.SemaphoreType.DMA,
            pltpu.SemaphoreType.DMA,
        ],
        **_kw(),
    )(ucols, srcp, dstp)


_MR = NP // 8


def _merge_body(parts_hbm, ucols_hbm, sm_hbm, d1_hbm, un_hbm, zc_hbm,
                pbuf, ubuf, sbuf, dbuf, sumb, obuf, zbuf):
    wid = _wid()
    col = lax.rem(wid, 4)
    nrng = wid // 4
    off = nrng * _MR
    for r in range(8):
        pltpu.sync_copy(parts_hbm.at[r, col, pl.ds(off, _MR)], pbuf.at[r])
    pltpu.sync_copy(ucols_hbm.at[col, pl.ds(off, _MR)], ubuf)
    pltpu.sync_copy(sm_hbm.at[pl.ds(off, _MR)], sbuf)

    @plsc.parallel_loop(0, _MR // 16, unroll=4)
    def _(i):
        sl = pl.ds(i * 16, 16)
        sm = ubuf[sl]
        for r in range(8):
            sm = sm + pbuf[r, sl]
        sumb[sl] = sm
        obuf[sl] = sbuf[sl] * sm
    pltpu.sync_copy(obuf, un_hbm.at[col, pl.ds(off, _MR)])

    @pl.when(col == 3)
    def _():
        pltpu.sync_copy(d1_hbm.at[pl.ds(off, _MR)], dbuf)

        @plsc.parallel_loop(0, _MR // 16, unroll=8)
        def _(i):
            sl = pl.ds(i * 16, 16)
            zbuf[sl] = dbuf[sl] * sumb[sl]
        pltpu.sync_copy(zbuf, zc_hbm.at[pl.ds(off, _MR)])


def _merge_call(parts, ucols, smain, d1):
    return pl.kernel(
        _merge_body,
        out_type=(jax.ShapeDtypeStruct((4, NP), _f32),
                  jax.ShapeDtypeStruct((NP,), _f32)),
        scratch_types=[
            pltpu.VMEM((8, _MR), _f32),
            pltpu.VMEM((_MR,), _f32),
            pltpu.VMEM((_MR,), _f32),
            pltpu.VMEM((_MR,), _f32),
            pltpu.VMEM((_MR,), _f32),
            pltpu.VMEM((_MR,), _f32),
            pltpu.VMEM((_MR,), _f32),
        ],
        **_kw(),
    )(parts, ucols, smain, d1)


_SR = NP // 32


def _seg_body(z3_hbm, zc1_hbm, zc2_hbm, bp_hbm, out_hbm,
              accf, bbuf, v0, v1, v2, v3, c1b, c2b):
    wid = _wid()
    _zero(accf, GP * GW)
    off = wid * _SR
    pltpu.sync_copy(bp_hbm.at[pl.ds(off, _SR)], bbuf)
    for k, vb in enumerate((v0, v1, v2, v3)):
        pltpu.sync_copy(z3_hbm.at[k, pl.ds(off, _SR)], vb)
    pltpu.sync_copy(zc1_hbm.at[pl.ds(off, _SR)], c1b)
    pltpu.sync_copy(zc2_hbm.at[pl.ds(off, _SR)], c2b)
    ones16 = jnp.full((16,), 1.0, _f32)

    @plsc.parallel_loop(0, _SR // 16, unroll=2)
    def _(i):
        sl = pl.ds(i * 16, 16)
        ix = bbuf[sl] * GW
        for cst, vb in ((0, v0), (1, v1), (2, v2), (3, v3),
                        (4, c1b), (5, c2b)):
            plsc.addupdate_scatter(accf, [ix + cst], vb[sl])
        plsc.addupdate_scatter(accf, [ix + 6], ones16)
    pltpu.sync_copy(accf, out_hbm.at[wid])


def _seg_call(z3c, zc1, zc2, bp):
    return pl.kernel(
        _seg_body,
        out_type=jax.ShapeDtypeStruct((32, GP * GW), _f32),
        scratch_types=[
            pltpu.VMEM((GP * GW,), _f32),
            pltpu.VMEM((_SR,), _i32),
            pltpu.VMEM((_SR,), _f32),
            pltpu.VMEM((_SR,), _f32),
            pltpu.VMEM((_SR,), _f32),
            pltpu.VMEM((_SR,), _f32),
            pltpu.VMEM((_SR,), _f32),
            pltpu.VMEM((_SR,), _f32),
        ],
        **_kw(),
    )(z3c, zc1, zc2, bp)


def _mseg_body(parts_hbm, ucols_hbm, d1_hbm, zc1_hbm, zc2_hbm, bp_hbm,
               out_hbm, accf, pb, ub, d1b, c1b, c2b, bb, sem):
    wid = _wid()
    off = wid * _SR
    _zero(accf, GP * GW)
    for r in range(8):
        for c in range(4):
            pltpu.async_copy(parts_hbm.at[r, c, pl.ds(off, _SR)],
                             pb.at[r * 4 + c], sem)
    for c in range(4):
        pltpu.async_copy(ucols_hbm.at[c, pl.ds(off, _SR)], ub.at[c], sem)
    pltpu.async_copy(d1_hbm.at[pl.ds(off, _SR)], d1b, sem)
    pltpu.async_copy(zc1_hbm.at[pl.ds(off, _SR)], c1b, sem)
    pltpu.async_copy(zc2_hbm.at[pl.ds(off, _SR)], c2b, sem)
    pltpu.async_copy(bp_hbm.at[pl.ds(off, _SR)], bb, sem)
    for r in range(8):
        for c in range(4):
            pltpu.make_async_copy(parts_hbm.at[r, c, pl.ds(off, _SR)],
                                  pb.at[r * 4 + c], sem).wait()
    for c in range(4):
        pltpu.make_async_copy(ucols_hbm.at[c, pl.ds(off, _SR)],
                              ub.at[c], sem).wait()
    pltpu.make_async_copy(d1_hbm.at[pl.ds(off, _SR)], d1b, sem).wait()
    pltpu.make_async_copy(zc1_hbm.at[pl.ds(off, _SR)], c1b, sem).wait()
    pltpu.make_async_copy(zc2_hbm.at[pl.ds(off, _SR)], c2b, sem).wait()
    pltpu.make_async_copy(bp_hbm.at[pl.ds(off, _SR)], bb, sem).wait()
    ones16 = jnp.full((16,), 1.0, _f32)

    @plsc.parallel_loop(0, _SR // 16, unroll=2)
    def _(i):
        sl = pl.ds(i * 16, 16)
        ixb = bb[sl] * GW
        dv = d1b[sl]
        for c in range(4):
            sm = ub[c, sl]
            for r in range(8):
                sm = sm + pb[r * 4 + c, sl]
            plsc.addupdate_scatter(accf, [ixb + c], dv * sm)
        plsc.addupdate_scatter(accf, [ixb + 4], c1b[sl])
        plsc.addupdate_scatter(accf, [ixb + 5], c2b[sl])
        plsc.addupdate_scatter(accf, [ixb + 6], ones16)

    pltpu.sync_copy(accf, out_hbm.at[wid])


def _mseg_call(parts, ucols, d1, zc1, zc2, bp):
    return pl.kernel(
        _mseg_body,
        out_type=jax.ShapeDtypeStruct((32, GP * GW), _f32),
        scratch_types=[
            pltpu.VMEM((GP * GW,), _f32),
            pltpu.VMEM((32, _SR), _f32),
            pltpu.VMEM((4, _SR), _f32),
            pltpu.VMEM((_SR,), _f32),
            pltpu.VMEM((_SR,), _f32),
            pltpu.VMEM((_SR,), _f32),
            pltpu.VMEM((_SR,), _i32),
            pltpu.SemaphoreType.DMA,
        ],
        **_kw(),
    )(parts, ucols, d1, zc1, zc2, bp)


_RB = NP // 8


def _prep_body(degp_ref, xt_ref, u1_ref, d1_ref, d2_ref):
    deg = jnp.sum(degp_ref[...], axis=0, keepdims=True) + 1.0
    dinv = lax.rsqrt(deg)
    dinv = dinv * (1.5 - 0.5 * deg * dinv * dinv)
    ones = jnp.ones_like(deg)
    u1_ref[...] = jnp.concatenate([xt_ref[...], ones], axis=0) * dinv
    d1_ref[...] = dinv
    d2_ref[...] = dinv * dinv


def _prep_call(degp, xt):
    return pl.pallas_call(
        _prep_body,
        grid=(NP // _RB,),
        in_specs=[
            pl.BlockSpec((32, _RB), lambda i: (0, i)),
            pl.BlockSpec((3, _RB), lambda i: (0, i)),
        ],
        out_specs=[
            pl.BlockSpec((4, _RB), lambda i: (0, i)),
            pl.BlockSpec((1, _RB), lambda i: (0, i)),
            pl.BlockSpec((1, _RB), lambda i: (0, i)),
        ],
        out_shape=[
            jax.ShapeDtypeStruct((4, NP), _f32),
            jax.ShapeDtypeStruct((1, NP), _f32),
            jax.ShapeDtypeStruct((1, NP), _f32),
        ],
    )(degp, xt)


def _final_body(segp_ref, w1_ref, b1_ref, w2_ref, b2_ref,
                w3_ref, b3_ref, out_ref):
    s = jnp.sum(segp_ref[...], axis=0)
    m3 = s[:512, 0:3]
    z1s = s[:512, 4:5]
    z2s = s[:512, 5:6]
    cnt = jnp.maximum(s[:512, 6:7], 1.0)
    w12 = jnp.dot(w1_ref[...], w2_ref[...], preferred_element_type=_f32, precision=lax.Precision.HIGHEST)
    w123 = jnp.dot(w12, w3_ref[...], preferred_element_type=_f32, precision=lax.Precision.HIGHEST)
    v1 = jnp.dot(jnp.dot(b1_ref[...], w2_ref[...],
                         preferred_element_type=_f32, precision=lax.Precision.HIGHEST),
                 w3_ref[...], preferred_element_type=_f32, precision=lax.Precision.HIGHEST)
    v2 = jnp.dot(b2_ref[...], w3_ref[...], preferred_element_type=_f32, precision=lax.Precision.HIGHEST)
    num = jnp.dot(m3, w123, preferred_element_type=_f32, precision=lax.Precision.HIGHEST)
    num = num + z2s * v1 + z1s * v2
    out_ref[...] = num / cnt + b3_ref[...]


def _final_call(segp, W1, b1, W2, b2, W3, b3):
    return pl.pallas_call(
        _final_body,
        out_shape=jax.ShapeDtypeStruct((G, 192), _f32),
    )(segp, W1, b1, W2, b2, W3, b3)


def kernel(node_features, edge_indices, batch_indices,
           W1, b1, W2, b2, W3, b3):
    epad = N + jnp.arange(EP - E, dtype=_i32) % (NP - N)
    srcp = jnp.concatenate([edge_indices[0], epad])
    dstp = jnp.concatenate([edge_indices[1], epad])
    xt = jnp.pad(node_features, ((0, NP - N), (0, 0))).T
    bp = jnp.pad(batch_indices, (0, NP - N), constant_values=G)

    degp = _deg_call(dstp)
    u1c, d1r, d2r = _prep_call(degp, xt)
    d1 = d1r.reshape(NP)
    d2 = d2r.reshape(NP)

    uc = u1c
    zcs = []
    for t in range(2):
        parts = _step_call(uc, srcp, dstp)
        uc, zc = _merge_call(parts, uc, d2, d1)
        zcs.append(zc)

    parts3 = _step_call(uc, srcp, dstp)
    segp = _mseg_call(parts3, uc, d1, zcs[0], zcs[1], bp).reshape(32, GP, GW)
    out = _final_call(segp, W1, b1.reshape(1, -1), W2,
                      b2.reshape(1, -1), W3, b3.reshape(1, -1))
    return out


# speedup vs baseline: 1.1845x; 1.0715x over previous
import jax
import jax.numpy as jnp
from jax import lax
from jax.experimental import pallas as pl
from jax.experimental.pallas import tpu as pltpu
from jax.experimental.pallas import tpu_sc as plsc

N = 50000
E = 800000
G = 512
NP = 50176
EP = 819200
EPR = EP // 8
EPW = EP // 32
K = 4096
KD = 5120
GP = 520
GW = 8

_f32 = jnp.float32
_i32 = jnp.int32

_SC_PARAMS = pltpu.CompilerParams(use_tc_tiling_on_sc=False,
                                  needs_layout_passes=False)


def _mesh():
    return plsc.VectorSubcoreMesh(core_axis_name="c", subcore_axis_name="s")


def _kw():
    return dict(mesh=_mesh(), compiler_params=_SC_PARAMS)


def _wid():
    return lax.axis_index("c") * 16 + lax.axis_index("s")


def _zero(buf, nwords):
    z = jnp.zeros((16,), _f32)

    @plsc.parallel_loop(0, nwords // 16, unroll=8)
    def _(i):
        buf[pl.ds(i * 16, 16)] = z


def _deg_body(dst_hbm, out_hbm, acc, b0, b1, s0, s1):
    wid = _wid()
    _zero(acc, NP)
    ones16 = jnp.full((16,), 1.0, _f32)
    base = wid * EPW
    nch = EPW // KD
    bufs = (b0, b1)
    sems = (s0, s1)

    def fire(ch):
        pltpu.async_copy(dst_hbm.at[pl.ds(base + ch * KD, KD)],
                         bufs[ch % 2], sems[ch % 2])

    fire(0)
    for ch in range(nch):
        buf, sem = bufs[ch % 2], sems[ch % 2]
        pltpu.make_async_copy(dst_hbm.at[pl.ds(base + ch * KD, KD)],
                              buf, sem).wait()
        if ch + 1 < nch:
            fire(ch + 1)

        @plsc.parallel_loop(0, KD // 16, unroll=16)
        def _(i):
            plsc.addupdate_scatter(acc, [buf[pl.ds(i * 16, 16)]], ones16)
    pltpu.sync_copy(acc, out_hbm.at[wid])


def _deg_call(dstp):
    return pl.kernel(
        _deg_body,
        out_type=jax.ShapeDtypeStruct((32, NP), _f32),
        scratch_types=[
            pltpu.VMEM((NP,), _f32),
            pltpu.VMEM((KD,), _i32),
            pltpu.VMEM((KD,), _i32),
            pltpu.SemaphoreType.DMA,
            pltpu.SemaphoreType.DMA,
        ],
        **_kw(),
    )(dstp)


def _step_body(ucols_hbm, src_hbm, dst_hbm, out_hbm,
               ucol, acc, sb0, db0, sb1, db1, ss0, sd0, ss1, sd1):
    wid = _wid()
    col = lax.rem(wid, 4)
    rng = wid // 4
    pltpu.async_copy(ucols_hbm.at[col], ucol, ss1)
    _zero(acc, NP)
    pltpu.make_async_copy(ucols_hbm.at[col], ucol, ss1).wait()
    base = rng * EPR
    nch = EPR // K
    sbufs = (sb0, sb1)
    dbufs = (db0, db1)
    ssems = (ss0, ss1)
    dsems = (sd0, sd1)

    def fire(ch):
        b = ch % 2
        pltpu.async_copy(src_hbm.at[pl.ds(base + ch * K, K)],
                         sbufs[b], ssems[b])
        pltpu.async_copy(dst_hbm.at[pl.ds(base + ch * K, K)],
                         dbufs[b], dsems[b])

    fire(0)
    for ch in range(nch):
        b = ch % 2
        pltpu.make_async_copy(src_hbm.at[pl.ds(base + ch * K, K)],
                              sbufs[b], ssems[b]).wait()
        pltpu.make_async_copy(dst_hbm.at[pl.ds(base + ch * K, K)],
                              dbufs[b], dsems[b]).wait()
        if ch + 1 < nch:
            fire(ch + 1)
        sbuf, dbuf = sbufs[b], dbufs[b]

        @plsc.parallel_loop(0, K // 16, unroll=16)
        def _(i):
            sl = pl.ds(i * 16, 16)
            g = plsc.load_gather(ucol, [sbuf[sl]])
            plsc.addupdate_scatter(acc, [dbuf[sl]], g)
    pltpu.sync_copy(acc, out_hbm.at[rng, col])


def _step_call(ucols, srcp, dstp):
    return pl.kernel(
        _step_body,
        out_type=jax.ShapeDtypeStruct((8, 4, NP), _f32),
        scratch_types=[
            pltpu.VMEM((NP,), _f32),
            pltpu.VMEM((NP,), _f32),
            pltpu.VMEM((K,), _i32),
            pltpu.VMEM((K,), _i32),
            pltpu.VMEM((K,), _i32),
            pltpu.VMEM((K,), _i32),
            pltpu.SemaphoreType.DMA,
            pltpu.SemaphoreType.DMA,
            pltpu.SemaphoreType.DMA,
            pltpu.SemaphoreType.DMA,
        ],
        **_kw(),
    )(ucols, srcp, dstp)


_MR = NP // 8


def _merge_body(parts_hbm, ucols_hbm, sm_hbm, d1_hbm, un_hbm, zc_hbm,
                pbuf, ubuf, sbuf, dbuf, sumb, obuf, zbuf, msem):
    wid = _wid()
    col = lax.rem(wid, 4)
    nrng = wid // 4
    off = nrng * _MR
    for r in range(8):
        pltpu.async_copy(parts_hbm.at[r, col, pl.ds(off, _MR)],
                         pbuf.at[r], msem)
    pltpu.async_copy(ucols_hbm.at[col, pl.ds(off, _MR)], ubuf, msem)
    pltpu.async_copy(sm_hbm.at[pl.ds(off, _MR)], sbuf, msem)
    for r in range(8):
        pltpu.make_async_copy(parts_hbm.at[r, col, pl.ds(off, _MR)],
                              pbuf.at[r], msem).wait()
    pltpu.make_async_copy(ucols_hbm.at[col, pl.ds(off, _MR)],
                          ubuf, msem).wait()
    pltpu.make_async_copy(sm_hbm.at[pl.ds(off, _MR)], sbuf, msem).wait()

    @plsc.parallel_loop(0, _MR // 16, unroll=4)
    def _(i):
        sl = pl.ds(i * 16, 16)
        sm = ubuf[sl]
        for r in range(8):
            sm = sm + pbuf[r, sl]
        sumb[sl] = sm
        obuf[sl] = sbuf[sl] * sm
    pltpu.sync_copy(obuf, un_hbm.at[col, pl.ds(off, _MR)])

    @pl.when(col == 3)
    def _():
        pltpu.sync_copy(d1_hbm.at[pl.ds(off, _MR)], dbuf)

        @plsc.parallel_loop(0, _MR // 16, unroll=8)
        def _(i):
            sl = pl.ds(i * 16, 16)
            zbuf[sl] = dbuf[sl] * sumb[sl]
        pltpu.sync_copy(zbuf, zc_hbm.at[pl.ds(off, _MR)])


def _merge_call(parts, ucols, smain, d1):
    return pl.kernel(
        _merge_body,
        out_type=(jax.ShapeDtypeStruct((4, NP), _f32),
                  jax.ShapeDtypeStruct((NP,), _f32)),
        scratch_types=[
            pltpu.VMEM((8, _MR), _f32),
            pltpu.VMEM((_MR,), _f32),
            pltpu.VMEM((_MR,), _f32),
            pltpu.VMEM((_MR,), _f32),
            pltpu.VMEM((_MR,), _f32),
            pltpu.VMEM((_MR,), _f32),
            pltpu.VMEM((_MR,), _f32),
            pltpu.SemaphoreType.DMA,
        ],
        **_kw(),
    )(parts, ucols, smain, d1)


_SR = NP // 32


def _seg_body(z3_hbm, zc1_hbm, zc2_hbm, bp_hbm, out_hbm,
              accf, bbuf, v0, v1, v2, v3, c1b, c2b):
    wid = _wid()
    _zero(accf, GP * GW)
    off = wid * _SR
    pltpu.sync_copy(bp_hbm.at[pl.ds(off, _SR)], bbuf)
    for k, vb in enumerate((v0, v1, v2, v3)):
        pltpu.sync_copy(z3_hbm.at[k, pl.ds(off, _SR)], vb)
    pltpu.sync_copy(zc1_hbm.at[pl.ds(off, _SR)], c1b)
    pltpu.sync_copy(zc2_hbm.at[pl.ds(off, _SR)], c2b)
    ones16 = jnp.full((16,), 1.0, _f32)

    @plsc.parallel_loop(0, _SR // 16, unroll=2)
    def _(i):
        sl = pl.ds(i * 16, 16)
        ix = bbuf[sl] * GW
        for cst, vb in ((0, v0), (1, v1), (2, v2), (3, v3),
                        (4, c1b), (5, c2b)):
            plsc.addupdate_scatter(accf, [ix + cst], vb[sl])
        plsc.addupdate_scatter(accf, [ix + 6], ones16)
    pltpu.sync_copy(accf, out_hbm.at[wid])


def _seg_call(z3c, zc1, zc2, bp):
    return pl.kernel(
        _seg_body,
        out_type=jax.ShapeDtypeStruct((32, GP * GW), _f32),
        scratch_types=[
            pltpu.VMEM((GP * GW,), _f32),
            pltpu.VMEM((_SR,), _i32),
            pltpu.VMEM((_SR,), _f32),
            pltpu.VMEM((_SR,), _f32),
            pltpu.VMEM((_SR,), _f32),
            pltpu.VMEM((_SR,), _f32),
            pltpu.VMEM((_SR,), _f32),
            pltpu.VMEM((_SR,), _f32),
        ],
        **_kw(),
    )(z3c, zc1, zc2, bp)


def _mseg_body(parts_hbm, ucols_hbm, d1_hbm, zc1_hbm, zc2_hbm, bp_hbm,
               out_hbm, accf, pb, ub, d1b, c1b, c2b, bb, sem):
    wid = _wid()
    off = wid * _SR
    _zero(accf, GP * GW)
    for r in range(8):
        for c in range(4):
            pltpu.async_copy(parts_hbm.at[r, c, pl.ds(off, _SR)],
                             pb.at[r * 4 + c], sem)
    for c in range(4):
        pltpu.async_copy(ucols_hbm.at[c, pl.ds(off, _SR)], ub.at[c], sem)
    pltpu.async_copy(d1_hbm.at[pl.ds(off, _SR)], d1b, sem)
    pltpu.async_copy(zc1_hbm.at[pl.ds(off, _SR)], c1b, sem)
    pltpu.async_copy(zc2_hbm.at[pl.ds(off, _SR)], c2b, sem)
    pltpu.async_copy(bp_hbm.at[pl.ds(off, _SR)], bb, sem)
    for r in range(8):
        for c in range(4):
            pltpu.make_async_copy(parts_hbm.at[r, c, pl.ds(off, _SR)],
                                  pb.at[r * 4 + c], sem).wait()
    for c in range(4):
        pltpu.make_async_copy(ucols_hbm.at[c, pl.ds(off, _SR)],
                              ub.at[c], sem).wait()
    pltpu.make_async_copy(d1_hbm.at[pl.ds(off, _SR)], d1b, sem).wait()
    pltpu.make_async_copy(zc1_hbm.at[pl.ds(off, _SR)], c1b, sem).wait()
    pltpu.make_async_copy(zc2_hbm.at[pl.ds(off, _SR)], c2b, sem).wait()
    pltpu.make_async_copy(bp_hbm.at[pl.ds(off, _SR)], bb, sem).wait()
    ones16 = jnp.full((16,), 1.0, _f32)

    @plsc.parallel_loop(0, _SR // 16, unroll=2)
    def _(i):
        sl = pl.ds(i * 16, 16)
        ixb = bb[sl] * GW
        dv = d1b[sl]
        for c in range(4):
            sm = ub[c, sl]
            for r in range(8):
                sm = sm + pb[r * 4 + c, sl]
            plsc.addupdate_scatter(accf, [ixb + c], dv * sm)
        plsc.addupdate_scatter(accf, [ixb + 4], c1b[sl])
        plsc.addupdate_scatter(accf, [ixb + 5], c2b[sl])
        plsc.addupdate_scatter(accf, [ixb + 6], ones16)

    pltpu.sync_copy(accf, out_hbm.at[wid])


def _mseg_call(parts, ucols, d1, zc1, zc2, bp):
    return pl.kernel(
        _mseg_body,
        out_type=jax.ShapeDtypeStruct((32, GP * GW), _f32),
        scratch_types=[
            pltpu.VMEM((GP * GW,), _f32),
            pltpu.VMEM((32, _SR), _f32),
            pltpu.VMEM((4, _SR), _f32),
            pltpu.VMEM((_SR,), _f32),
            pltpu.VMEM((_SR,), _f32),
            pltpu.VMEM((_SR,), _f32),
            pltpu.VMEM((_SR,), _i32),
            pltpu.SemaphoreType.DMA,
        ],
        **_kw(),
    )(parts, ucols, d1, zc1, zc2, bp)


_RB = NP // 8


def _prep_body(degp_ref, xt_ref, u1_ref, d1_ref, d2_ref):
    deg = jnp.sum(degp_ref[...], axis=0, keepdims=True) + 1.0
    dinv = lax.rsqrt(deg)
    dinv = dinv * (1.5 - 0.5 * deg * dinv * dinv)
    ones = jnp.ones_like(deg)
    u1_ref[...] = jnp.concatenate([xt_ref[...], ones], axis=0) * dinv
    d1_ref[...] = dinv
    d2_ref[...] = dinv * dinv


def _prep_call(degp, xt):
    return pl.pallas_call(
        _prep_body,
        grid=(NP // _RB,),
        in_specs=[
            pl.BlockSpec((32, _RB), lambda i: (0, i)),
            pl.BlockSpec((3, _RB), lambda i: (0, i)),
        ],
        out_specs=[
            pl.BlockSpec((4, _RB), lambda i: (0, i)),
            pl.BlockSpec((1, _RB), lambda i: (0, i)),
            pl.BlockSpec((1, _RB), lambda i: (0, i)),
        ],
        out_shape=[
            jax.ShapeDtypeStruct((4, NP), _f32),
            jax.ShapeDtypeStruct((1, NP), _f32),
            jax.ShapeDtypeStruct((1, NP), _f32),
        ],
    )(degp, xt)


def _final_body(segp_ref, w1_ref, b1_ref, w2_ref, b2_ref,
                w3_ref, b3_ref, out_ref):
    s = jnp.sum(segp_ref[...], axis=0)
    m3 = s[:512, 0:3]
    z1s = s[:512, 4:5]
    z2s = s[:512, 5:6]
    cnt = jnp.maximum(s[:512, 6:7], 1.0)
    w12 = jnp.dot(w1_ref[...], w2_ref[...], preferred_element_type=_f32, precision=lax.Precision.HIGHEST)
    w123 = jnp.dot(w12, w3_ref[...], preferred_element_type=_f32, precision=lax.Precision.HIGHEST)
    v1 = jnp.dot(jnp.dot(b1_ref[...], w2_ref[...],
                         preferred_element_type=_f32, precision=lax.Precision.HIGHEST),
                 w3_ref[...], preferred_element_type=_f32, precision=lax.Precision.HIGHEST)
    v2 = jnp.dot(b2_ref[...], w3_ref[...], preferred_element_type=_f32, precision=lax.Precision.HIGHEST)
    num = jnp.dot(m3, w123, preferred_element_type=_f32, precision=lax.Precision.HIGHEST)
    num = num + z2s * v1 + z1s * v2
    out_ref[...] = num / cnt + b3_ref[...]


def _final_call(segp, W1, b1, W2, b2, W3, b3):
    return pl.pallas_call(
        _final_body,
        out_shape=jax.ShapeDtypeStruct((G, 192), _f32),
    )(segp, W1, b1, W2, b2, W3, b3)


def kernel(node_features, edge_indices, batch_indices,
           W1, b1, W2, b2, W3, b3):
    epad = N + jnp.arange(EP - E, dtype=_i32) % (NP - N)
    srcp = jnp.concatenate([edge_indices[0], epad])
    dstp = jnp.concatenate([edge_indices[1], epad])
    xt = jnp.pad(node_features, ((0, NP - N), (0, 0))).T
    bp = jnp.pad(batch_indices, (0, NP - N), constant_values=G)

    degp = _deg_call(dstp)
    u1c, d1r, d2r = _prep_call(degp, xt)
    d1 = d1r.reshape(NP)
    d2 = d2r.reshape(NP)

    uc = u1c
    zcs = []
    for t in range(2):
        parts = _step_call(uc, srcp, dstp)
        uc, zc = _merge_call(parts, uc, d2, d1)
        zcs.append(zc)

    parts3 = _step_call(uc, srcp, dstp)
    segp = _mseg_call(parts3, uc, d1, zcs[0], zcs[1], bp).reshape(32, GP, GW)
    out = _final_call(segp, W1, b1.reshape(1, -1), W2,
                      b2.reshape(1, -1), W3, b3.reshape(1, -1))
    return out
